# baseline (speedup 1.0000x reference)
import functools

import jax
import jax.numpy as jnp
from jax import lax
from jax.experimental import pallas as pl
from jax.experimental.pallas import tpu as pltpu
from jax.experimental.pallas import tpu_sc as plsc

N = 10000
E = 320000
D = 128
C = 64
G = 64
NOUT = 10

NC = 2
NS = 16
NW = NC * NS
EPW = E // NW
CH = 1000
NCH = EPW // CH
NP = 10240
RPT = NP // NS
NPK = NP // 2

NB = 5
BM = NP // NB
HB = BM // 2


def _edge_body(u_hbm, src_hbm, dst_hbm, zero_hbm, out_hbm,
               src_v, dst_v, rows_a, rows_b, stage_v, acc_sh,
               sem_a, sem_b):
    cid = lax.axis_index("c")
    sid = lax.axis_index("s")
    wid = sid * NC + cid

    pltpu.sync_copy(zero_hbm.at[pl.ds(sid * RPT, RPT)], stage_v)
    pltpu.sync_copy(stage_v, acc_sh.at[pl.ds(sid * RPT, RPT)])

    pltpu.sync_copy(src_hbm.at[wid], src_v)
    pltpu.sync_copy(dst_hbm.at[wid], dst_v)

    pltpu.async_copy(u_hbm.at[src_v.at[0]], rows_a, sem_a)
    plsc.subcore_barrier()

    def body(j, carry):
        c0 = 2 * j
        pltpu.async_copy(u_hbm.at[src_v.at[c0 + 1]], rows_b, sem_b)
        pltpu.make_async_copy(u_hbm.at[src_v.at[c0]], rows_a, sem_a).wait()
        pltpu.sync_copy(rows_a, acc_sh.at[dst_v.at[c0]], add=True)
        pltpu.async_copy(u_hbm.at[src_v.at[c0 + 2]], rows_a, sem_a)
        pltpu.make_async_copy(u_hbm.at[src_v.at[c0 + 1]], rows_b, sem_b).wait()
        pltpu.sync_copy(rows_b, acc_sh.at[dst_v.at[c0 + 1]], add=True)
        return carry

    lax.fori_loop(0, NCH // 2 - 1, body, 0)

    pltpu.async_copy(u_hbm.at[src_v.at[NCH - 1]], rows_b, sem_b)
    pltpu.make_async_copy(u_hbm.at[src_v.at[NCH - 2]], rows_a, sem_a).wait()
    pltpu.sync_copy(rows_a, acc_sh.at[dst_v.at[NCH - 2]], add=True)
    pltpu.make_async_copy(u_hbm.at[src_v.at[NCH - 1]], rows_b, sem_b).wait()
    pltpu.sync_copy(rows_b, acc_sh.at[dst_v.at[NCH - 1]], add=True)

    plsc.subcore_barrier()

    pltpu.sync_copy(acc_sh.at[pl.ds(sid * RPT, RPT)], stage_v)
    pltpu.sync_copy(stage_v, out_hbm.at[cid, pl.ds(sid * RPT, RPT)])


@functools.lru_cache(maxsize=1)
def _build_edge_agg():
    return pl.kernel(
        _edge_body,
        mesh=plsc.VectorSubcoreMesh(core_axis_name="c", subcore_axis_name="s",
                                    num_cores=NC, num_subcores=NS),
        out_type=jax.ShapeDtypeStruct((NC, NP, C), jnp.bfloat16),
        scratch_types=[
            pltpu.VMEM((NCH, CH), jnp.int32),
            pltpu.VMEM((NCH, CH), jnp.int32),
            pltpu.VMEM((CH, C), jnp.bfloat16),
            pltpu.VMEM((CH, C), jnp.bfloat16),
            pltpu.VMEM((RPT, C), jnp.bfloat16),
            pltpu.VMEM_SHARED((NP, C), jnp.bfloat16),
            pltpu.SemaphoreType.DMA,
            pltpu.SemaphoreType.DMA,
        ],
        compiler_params=pltpu.CompilerParams(use_tc_tiling_on_sc=False),
    )


def _edge_agg(ub, srcr, dstr, zeros):
    u_lin = ub.reshape(NP, C)
    return _build_edge_agg()(u_lin, srcr, dstr, zeros).reshape(NC, NPK, D)


def _pack(v):
    return jnp.concatenate([v[:HB], v[HB:]], axis=1).astype(jnp.bfloat16)


def _proj_body(x_ref, w_ref, us_ref, ub_ref):
    u = jnp.dot(x_ref[...], w_ref[...], preferred_element_type=jnp.float32)
    us_ref[...] = u
    ub_ref[...] = _pack(u)


def _proj(x, w):
    return pl.pallas_call(
        _proj_body,
        grid=(NB,),
        in_specs=[
            pl.BlockSpec((BM, D), lambda i: (i, 0)),
            pl.BlockSpec(w.shape, lambda i: (0, 0)),
        ],
        out_specs=[pl.BlockSpec((BM, C), lambda i: (i, 0)),
                   pl.BlockSpec((HB, D), lambda i: (i, 0))],
        out_shape=[jax.ShapeDtypeStruct((NP, C), jnp.float32),
                   jax.ShapeDtypeStruct((NPK, D), jnp.bfloat16)],
    )(x, w)


def _z_block(us_ref, agg_ref, b0):
    a = agg_ref[0].astype(jnp.float32) + agg_ref[1].astype(jnp.float32)
    agg = jnp.concatenate([a[:, 0:C], a[:, C:D]], axis=0)
    return us_ref[...] + agg + b0


def _mlp(z, w1, b1, w2, b2):
    t = jnp.maximum(z, 0.0)
    t = jnp.maximum(
        jnp.dot(t, w1, preferred_element_type=jnp.float32) + b1, 0.0)
    return jnp.dot(t, w2, preferred_element_type=jnp.float32) + b2


def _mlp_body(us_ref, agg_ref, b0_ref, w1_ref, b1_ref, w2_ref, b2_ref, wn_ref,
              uso_ref, ubo_ref):
    z = _z_block(us_ref, agg_ref, b0_ref[...])
    h = _mlp(z, w1_ref[...], b1_ref[...], w2_ref[...], b2_ref[...])
    un = jnp.dot(h, wn_ref[...], preferred_element_type=jnp.float32)
    uso_ref[...] = un
    ubo_ref[...] = _pack(un)


def _mlp_proj(us, aggp, b0, w1, b1, w2, b2, wn):
    full = lambda a: pl.BlockSpec(a.shape, lambda i: (0,) * a.ndim)
    return pl.pallas_call(
        _mlp_body,
        grid=(NB,),
        in_specs=[
            pl.BlockSpec((BM, C), lambda i: (i, 0)),
            pl.BlockSpec((NC, HB, D), lambda i: (0, i, 0)),
            full(b0), full(w1), full(b1), full(w2), full(b2), full(wn),
        ],
        out_specs=[pl.BlockSpec((BM, C), lambda i: (i, 0)),
                   pl.BlockSpec((HB, D), lambda i: (i, 0))],
        out_shape=[jax.ShapeDtypeStruct((NP, C), jnp.float32),
                   jax.ShapeDtypeStruct((NPK, D), jnp.bfloat16)],
    )(us, aggp, b0, w1, b1, w2, b2, wn)


def _head_body(us_ref, agg_ref, gid_ref, b0_ref, w1_ref, b1_ref, w2_ref,
               b2_ref, d1w_ref, d1b_ref, d2w_ref, d2b_ref, o_ref,
               pool_acc, cnt_acc):
    i = pl.program_id(0)

    z = _z_block(us_ref, agg_ref, b0_ref[...])
    h = _mlp(z, w1_ref[...], b1_ref[...], w2_ref[...], b2_ref[...])

    ids = gid_ref[0]
    giota = lax.broadcasted_iota(jnp.int32, (1, G), 1)
    oh = jnp.where(ids == giota, 1.0, 0.0)
    h = jnp.where(ids < G, h, 0.0)
    dn = (((0,), (0,)), ((), ()))
    psum = lax.dot_general(oh, h, dn, preferred_element_type=jnp.float32)
    ones = jnp.ones((BM, 1), jnp.float32)
    csum = lax.dot_general(oh, ones, dn, preferred_element_type=jnp.float32)

    @pl.when(i == 0)
    def _():
        pool_acc[...] = psum
        cnt_acc[...] = csum

    @pl.when(i > 0)
    def _():
        pool_acc[...] += psum
        cnt_acc[...] += csum

    @pl.when(i == NB - 1)
    def _():
        pooled = pool_acc[...] / jnp.maximum(cnt_acc[...], 1.0)
        r = jnp.maximum(
            jnp.dot(pooled, d1w_ref[...], preferred_element_type=jnp.float32)
            + d1b_ref[...], 0.0)
        logits = jnp.dot(r, d2w_ref[...],
                         preferred_element_type=jnp.float32) + d2b_ref[...]
        m = jnp.max(logits, axis=-1, keepdims=True)
        e = jnp.exp(logits - m)
        o_ref[...] = e / jnp.sum(e, axis=-1, keepdims=True)


def _mlp_pool_head(us, aggp, gid3, b0, w1, b1, w2, b2, d1w, d1b, d2w, d2b):
    full = lambda a: pl.BlockSpec(a.shape, lambda i: (0,) * a.ndim)
    return pl.pallas_call(
        _head_body,
        grid=(NB,),
        in_specs=[
            pl.BlockSpec((BM, C), lambda i: (i, 0)),
            pl.BlockSpec((NC, HB, D), lambda i: (0, i, 0)),
            pl.BlockSpec((1, BM, 1), lambda i: (i, 0, 0)),
            full(b0), full(w1), full(b1), full(w2), full(b2),
            full(d1w), full(d1b), full(d2w), full(d2b),
        ],
        out_specs=pl.BlockSpec((G, NOUT), lambda i: (0, 0)),
        out_shape=jax.ShapeDtypeStruct((G, NOUT), jnp.float32),
        scratch_shapes=[
            pltpu.VMEM((G, C), jnp.float32),
            pltpu.VMEM((G, 1), jnp.float32),
        ],
    )(us, aggp, gid3, b0, w1, b1, w2, b2, d1w, d1b, d2w, d2b)


def _perm(n):
    return (n & ~jnp.int32(BM - 1)) | ((n & (HB - 1)) << 1) | ((n >> 10) & 1)


def kernel(x, edge_index, graph_ids, params):
    p = params
    srcr = _perm(edge_index[0]).reshape(NW, NCH, CH)
    dstr = _perm(edge_index[1]).reshape(NW, NCH, CH)
    zeros = jnp.zeros((NP, C), jnp.bfloat16)
    gid3 = jnp.concatenate([graph_ids, jnp.full((NP - N,), G, jnp.int32)]
                           ).reshape(NB, BM, 1)

    row = lambda b: b.reshape(1, -1)

    us, ub = _proj(x, p['conv0_W0'])
    for l in range(2):
        aggp = _edge_agg(ub, srcr, dstr, zeros)
        us, ub = _mlp_proj(us, aggp,
                           row(p['conv%d_b0' % l]), p['conv%d_W1' % l],
                           row(p['conv%d_b1' % l]), p['conv%d_W2' % l],
                           row(p['conv%d_b2' % l]), p['conv%d_W0' % (l + 1)])
    aggp = _edge_agg(ub, srcr, dstr, zeros)
    return _mlp_pool_head(us, aggp, gid3,
                          row(p['conv2_b0']), p['conv2_W1'],
                          row(p['conv2_b1']), p['conv2_W2'],
                          row(p['conv2_b2']),
                          p['dense1_W'], row(p['dense1_b']),
                          p['dense2_W'], row(p['dense2_b']))


# speedup vs baseline: 1.0172x; 1.0172x over previous
import functools

import jax
import jax.numpy as jnp
from jax import lax
from jax.experimental import pallas as pl
from jax.experimental.pallas import tpu as pltpu
from jax.experimental.pallas import tpu_sc as plsc

N = 10000
E = 320000
D = 128
C = 64
G = 64
NOUT = 10

NC = 2
NS = 16
NW = NC * NS
EPW = E // NW
CH = 1000
NCH = EPW // CH
NP = 10240
RPT = NP // NS
NPK = NP // 2

NB = 5
BM = NP // NB
HB = BM // 2


def _edge_body(u_hbm, src_hbm, dst_hbm, zero_hbm, out_hbm,
               src_v, dst_v, rows_a, rows_b, stage_v, acc_sh,
               sem_a, sem_b):
    cid = lax.axis_index("c")
    sid = lax.axis_index("s")
    wid = sid * NC + cid

    pltpu.sync_copy(zero_hbm.at[pl.ds(sid * RPT, RPT)], stage_v)
    pltpu.sync_copy(stage_v, acc_sh.at[pl.ds(sid * RPT, RPT)])

    pltpu.sync_copy(src_hbm.at[wid], src_v)
    pltpu.sync_copy(dst_hbm.at[wid], dst_v)

    pltpu.async_copy(u_hbm.at[src_v.at[0]], rows_a, sem_a)
    plsc.subcore_barrier()

    def body(j, carry):
        c0 = 2 * j
        pltpu.async_copy(u_hbm.at[src_v.at[c0 + 1]], rows_b, sem_b)
        pltpu.make_async_copy(u_hbm.at[src_v.at[c0]], rows_a, sem_a).wait()
        pltpu.sync_copy(rows_a, acc_sh.at[dst_v.at[c0]], add=True)
        pltpu.async_copy(u_hbm.at[src_v.at[c0 + 2]], rows_a, sem_a)
        pltpu.make_async_copy(u_hbm.at[src_v.at[c0 + 1]], rows_b, sem_b).wait()
        pltpu.sync_copy(rows_b, acc_sh.at[dst_v.at[c0 + 1]], add=True)
        return carry

    lax.fori_loop(0, NCH // 2 - 1, body, 0)

    pltpu.async_copy(u_hbm.at[src_v.at[NCH - 1]], rows_b, sem_b)
    pltpu.make_async_copy(u_hbm.at[src_v.at[NCH - 2]], rows_a, sem_a).wait()
    pltpu.sync_copy(rows_a, acc_sh.at[dst_v.at[NCH - 2]], add=True)
    pltpu.make_async_copy(u_hbm.at[src_v.at[NCH - 1]], rows_b, sem_b).wait()
    pltpu.sync_copy(rows_b, acc_sh.at[dst_v.at[NCH - 1]], add=True)

    plsc.subcore_barrier()

    pltpu.sync_copy(acc_sh.at[pl.ds(sid * RPT, RPT)], stage_v)
    pltpu.sync_copy(stage_v, out_hbm.at[cid, pl.ds(sid * RPT, RPT)])


@functools.lru_cache(maxsize=1)
def _build_edge_agg():
    return pl.kernel(
        _edge_body,
        mesh=plsc.VectorSubcoreMesh(core_axis_name="c", subcore_axis_name="s",
                                    num_cores=NC, num_subcores=NS),
        out_type=jax.ShapeDtypeStruct((NC, NP, C), jnp.bfloat16),
        scratch_types=[
            pltpu.VMEM((NCH, CH), jnp.int32),
            pltpu.VMEM((NCH, CH), jnp.int32),
            pltpu.VMEM((CH, C), jnp.bfloat16),
            pltpu.VMEM((CH, C), jnp.bfloat16),
            pltpu.VMEM((RPT, C), jnp.bfloat16),
            pltpu.VMEM_SHARED((NP, C), jnp.bfloat16),
            pltpu.SemaphoreType.DMA,
            pltpu.SemaphoreType.DMA,
        ],
        compiler_params=pltpu.CompilerParams(use_tc_tiling_on_sc=False),
    )


def _edge_agg(ub, srcr, dstr, zeros):
    u_lin = ub.reshape(NP, C)
    return _build_edge_agg()(u_lin, srcr, dstr, zeros).reshape(NC, NPK, D)


def _pack(v):
    return jnp.concatenate([v[:HB], v[HB:]], axis=1)


def _unpack(v):
    return jnp.concatenate([v[:, 0:C], v[:, C:D]], axis=0)


def _proj_body(x_ref, w_ref, us_ref, ub_ref):
    u = jnp.dot(x_ref[...], w_ref[...], preferred_element_type=jnp.float32)
    up = _pack(u)
    us_ref[...] = up
    ub_ref[...] = up.astype(jnp.bfloat16)


def _proj(x, w):
    return pl.pallas_call(
        _proj_body,
        grid=(NB,),
        in_specs=[
            pl.BlockSpec((BM, D), lambda i: (i, 0)),
            pl.BlockSpec(w.shape, lambda i: (0, 0)),
        ],
        out_specs=[pl.BlockSpec((HB, D), lambda i: (i, 0)),
                   pl.BlockSpec((HB, D), lambda i: (i, 0))],
        out_shape=[jax.ShapeDtypeStruct((NPK, D), jnp.float32),
                   jax.ShapeDtypeStruct((NPK, D), jnp.bfloat16)],
    )(x, w)


def _z_block(us_ref, agg_ref, b0):
    a = agg_ref[0].astype(jnp.float32) + agg_ref[1].astype(jnp.float32)
    return _unpack(us_ref[...] + a) + b0


def _mlp(z, w1, b1, w2, b2):
    t = jnp.maximum(z, 0.0)
    t = jnp.maximum(
        jnp.dot(t, w1, preferred_element_type=jnp.float32) + b1, 0.0)
    return jnp.dot(t, w2, preferred_element_type=jnp.float32) + b2


def _mlp_body(us_ref, agg_ref, b0_ref, w1_ref, b1_ref, w2_ref, b2_ref, wn_ref,
              uso_ref, ubo_ref):
    z = _z_block(us_ref, agg_ref, b0_ref[...])
    h = _mlp(z, w1_ref[...], b1_ref[...], w2_ref[...], b2_ref[...])
    un = jnp.dot(h, wn_ref[...], preferred_element_type=jnp.float32)
    unp = _pack(un)
    uso_ref[...] = unp
    ubo_ref[...] = unp.astype(jnp.bfloat16)


def _mlp_proj(us, aggp, b0, w1, b1, w2, b2, wn):
    full = lambda a: pl.BlockSpec(a.shape, lambda i: (0,) * a.ndim)
    return pl.pallas_call(
        _mlp_body,
        grid=(NB,),
        in_specs=[
            pl.BlockSpec((HB, D), lambda i: (i, 0)),
            pl.BlockSpec((NC, HB, D), lambda i: (0, i, 0)),
            full(b0), full(w1), full(b1), full(w2), full(b2), full(wn),
        ],
        out_specs=[pl.BlockSpec((HB, D), lambda i: (i, 0)),
                   pl.BlockSpec((HB, D), lambda i: (i, 0))],
        out_shape=[jax.ShapeDtypeStruct((NPK, D), jnp.float32),
                   jax.ShapeDtypeStruct((NPK, D), jnp.bfloat16)],
    )(us, aggp, b0, w1, b1, w2, b2, wn)


def _head_body(us_ref, agg_ref, gid_ref, b0_ref, w1_ref, b1_ref, w2_ref,
               b2_ref, d1w_ref, d1b_ref, d2w_ref, d2b_ref, o_ref,
               pool_acc, cnt_acc):
    i = pl.program_id(0)

    z = _z_block(us_ref, agg_ref, b0_ref[...])
    h = _mlp(z, w1_ref[...], b1_ref[...], w2_ref[...], b2_ref[...])

    ids = gid_ref[0]
    giota = lax.broadcasted_iota(jnp.int32, (1, G), 1)
    oh = jnp.where(ids == giota, 1.0, 0.0)
    h = jnp.where(ids < G, h, 0.0)
    dn = (((0,), (0,)), ((), ()))
    psum = lax.dot_general(oh, h, dn, preferred_element_type=jnp.float32)
    ones = jnp.ones((BM, 1), jnp.float32)
    csum = lax.dot_general(oh, ones, dn, preferred_element_type=jnp.float32)

    @pl.when(i == 0)
    def _():
        pool_acc[...] = psum
        cnt_acc[...] = csum

    @pl.when(i > 0)
    def _():
        pool_acc[...] += psum
        cnt_acc[...] += csum

    @pl.when(i == NB - 1)
    def _():
        pooled = pool_acc[...] / jnp.maximum(cnt_acc[...], 1.0)
        r = jnp.maximum(
            jnp.dot(pooled, d1w_ref[...], preferred_element_type=jnp.float32)
            + d1b_ref[...], 0.0)
        logits = jnp.dot(r, d2w_ref[...],
                         preferred_element_type=jnp.float32) + d2b_ref[...]
        m = jnp.max(logits, axis=-1, keepdims=True)
        e = jnp.exp(logits - m)
        o_ref[...] = e / jnp.sum(e, axis=-1, keepdims=True)


def _mlp_pool_head(us, aggp, gid3, b0, w1, b1, w2, b2, d1w, d1b, d2w, d2b):
    full = lambda a: pl.BlockSpec(a.shape, lambda i: (0,) * a.ndim)
    return pl.pallas_call(
        _head_body,
        grid=(NB,),
        in_specs=[
            pl.BlockSpec((HB, D), lambda i: (i, 0)),
            pl.BlockSpec((NC, HB, D), lambda i: (0, i, 0)),
            pl.BlockSpec((1, BM, 1), lambda i: (i, 0, 0)),
            full(b0), full(w1), full(b1), full(w2), full(b2),
            full(d1w), full(d1b), full(d2w), full(d2b),
        ],
        out_specs=pl.BlockSpec((G, NOUT), lambda i: (0, 0)),
        out_shape=jax.ShapeDtypeStruct((G, NOUT), jnp.float32),
        scratch_shapes=[
            pltpu.VMEM((G, C), jnp.float32),
            pltpu.VMEM((G, 1), jnp.float32),
        ],
    )(us, aggp, gid3, b0, w1, b1, w2, b2, d1w, d1b, d2w, d2b)


def _perm(n):
    return (n & ~jnp.int32(BM - 1)) | ((n & (HB - 1)) << 1) | ((n >> 10) & 1)


def kernel(x, edge_index, graph_ids, params):
    p = params
    srcr = _perm(edge_index[0]).reshape(NW, NCH, CH)
    dstr = _perm(edge_index[1]).reshape(NW, NCH, CH)
    zeros = jnp.zeros((NP, C), jnp.bfloat16)
    gid3 = jnp.concatenate([graph_ids, jnp.full((NP - N,), G, jnp.int32)]
                           ).reshape(NB, BM, 1)

    row = lambda b: b.reshape(1, -1)

    us, ub = _proj(x, p['conv0_W0'])
    for l in range(2):
        aggp = _edge_agg(ub, srcr, dstr, zeros)
        us, ub = _mlp_proj(us, aggp,
                           row(p['conv%d_b0' % l]), p['conv%d_W1' % l],
                           row(p['conv%d_b1' % l]), p['conv%d_W2' % l],
                           row(p['conv%d_b2' % l]), p['conv%d_W0' % (l + 1)])
    aggp = _edge_agg(ub, srcr, dstr, zeros)
    return _mlp_pool_head(us, aggp, gid3,
                          row(p['conv2_b0']), p['conv2_W1'],
                          row(p['conv2_b1']), p['conv2_W2'],
                          row(p['conv2_b2']),
                          p['dense1_W'], row(p['dense1_b']),
                          p['dense2_W'], row(p['dense2_b']))
